# baseline (speedup 1.0000x reference)
import functools

import jax
import jax.numpy as jnp
from jax.experimental import pallas as pl
from jax.experimental.pallas import tpu as pltpu


def _main_body(x_ref, wg1t_ref, wot_ref, wvt_ref, wg2t_ref, ke_ref, bg_ref,
               bo_ref, gamma_ref, beta_ref, out_ref, v_ref, c_ref):
    i = pl.program_id(0)

    @pl.when(i == 0)
    def _():
        v = jnp.dot(ke_ref[...].astype(jnp.bfloat16), wvt_ref[...],
                    preferred_element_type=jnp.float32)
        c = jnp.dot(v.astype(jnp.bfloat16), wg2t_ref[...],
                    preferred_element_type=jnp.float32) + bg_ref[...]
        v_ref[...] = v
        c_ref[...] = c

    tm = x_ref.shape[0]
    half = tm // 2

    def chain(x):
        xb = x.astype(jnp.bfloat16)
        g = jax.nn.sigmoid(
            jnp.dot(xb, wg1t_ref[...],
                    preferred_element_type=jnp.float32)
            + c_ref[...])
        vb = v_ref[...].astype(jnp.bfloat16)
        eb = xb + g.astype(jnp.bfloat16) * (vb - xb)
        p = jnp.dot(eb, wot_ref[...],
                    preferred_element_type=jnp.float32) + bo_ref[...]
        h = p.shape[-1]
        mu = jnp.sum(p, axis=-1, keepdims=True) * (1.0 / h)
        msq = jnp.sum(p * p, axis=-1, keepdims=True) * (1.0 / h)
        var = msq - mu * mu
        scale = (jax.lax.rsqrt(var + 1e-5) * gamma_ref[...]).astype(jnp.bfloat16)
        shift = (beta_ref[...] - mu * jax.lax.rsqrt(var + 1e-5)
                 * gamma_ref[...])
        ln = p.astype(jnp.bfloat16) * scale + shift.astype(jnp.bfloat16)
        return ln.astype(jnp.float32) + x

    out_ref[:half, :] = chain(x_ref[:half, :])
    out_ref[half:, :] = chain(x_ref[half:, :])


@functools.partial(jax.jit, static_argnames=("interpret",))
def kernel(sequence, knowledge_embeddings, Wq, Wk, Wv, Wg, bg, Wo, bo,
           ln_gamma, ln_beta, interpret=False):
    B, S, H = sequence.shape
    N = B * S
    TM = 512
    x = sequence.reshape(N, H)

    ke0 = knowledge_embeddings[:1]
    wvt = Wv.T.astype(jnp.bfloat16)
    wg2t = Wg[:, H:].T.astype(jnp.bfloat16)
    wg1t = Wg[:, :H].T.astype(jnp.bfloat16)
    wot = Wo.T.astype(jnp.bfloat16)
    bg2 = bg.reshape(1, H)
    bo2 = bo.reshape(1, H)
    gamma2 = ln_gamma.reshape(1, H)
    beta2 = ln_beta.reshape(1, H)

    full = lambda i: (0, 0)
    out = pl.pallas_call(
        _main_body,
        grid=(N // TM,),
        in_specs=[
            pl.BlockSpec((TM, H), lambda i: (i, 0)),
            pl.BlockSpec((H, H), full),
            pl.BlockSpec((H, H), full),
            pl.BlockSpec((H, H), full),
            pl.BlockSpec((H, H), full),
            pl.BlockSpec((1, H), full),
            pl.BlockSpec((1, H), full),
            pl.BlockSpec((1, H), full),
            pl.BlockSpec((1, H), full),
            pl.BlockSpec((1, H), full),
        ],
        out_specs=pl.BlockSpec((TM, H), lambda i: (i, 0)),
        out_shape=jax.ShapeDtypeStruct((N, H), jnp.float32),
        scratch_shapes=[
            pltpu.VMEM((1, H), jnp.float32),
            pltpu.VMEM((1, H), jnp.float32),
        ],
        interpret=interpret,
    )(x, wg1t, wot, wvt, wg2t, ke0, bg2, bo2, gamma2, beta2)

    return out.reshape(B, S, H)


# speedup vs baseline: 1.0097x; 1.0097x over previous
import functools

import jax
import jax.numpy as jnp
from jax.experimental import pallas as pl
from jax.experimental.pallas import tpu as pltpu


def _main_body(x_ref, wg1t_ref, wot_ref, wvt_ref, wg2t_ref, ke_ref, bg_ref,
               bo_ref, gamma_ref, beta_ref, out_ref, v_ref, c_ref):
    i = pl.program_id(0)

    @pl.when(i == 0)
    def _():
        v = jnp.dot(ke_ref[...].astype(jnp.bfloat16), wvt_ref[...],
                    preferred_element_type=jnp.float32)
        c = jnp.dot(v.astype(jnp.bfloat16), wg2t_ref[...],
                    preferred_element_type=jnp.float32) + bg_ref[...]
        v_ref[...] = v
        c_ref[...] = c

    tm = x_ref.shape[0]
    half = tm // 2

    def chain(x):
        xb = x.astype(jnp.bfloat16)
        g = jax.nn.sigmoid(
            jnp.dot(xb, wg1t_ref[...],
                    preferred_element_type=jnp.float32)
            + c_ref[...])
        vb = v_ref[...].astype(jnp.bfloat16)
        eb = xb + g.astype(jnp.bfloat16) * (vb - xb)
        p = jnp.dot(eb, wot_ref[...],
                    preferred_element_type=jnp.float32) + bo_ref[...]
        h = p.shape[-1]
        mu = jnp.sum(p, axis=-1, keepdims=True) * (1.0 / h)
        msq = jnp.sum(p * p, axis=-1, keepdims=True) * (1.0 / h)
        var = msq - mu * mu
        ln = ((p - mu) * jax.lax.rsqrt(var + 1e-5) * gamma_ref[...]
              + beta_ref[...])
        return ln + x

    out_ref[...] = chain(x_ref[...])


@functools.partial(jax.jit, static_argnames=("interpret",))
def kernel(sequence, knowledge_embeddings, Wq, Wk, Wv, Wg, bg, Wo, bo,
           ln_gamma, ln_beta, interpret=False):
    B, S, H = sequence.shape
    N = B * S
    TM = 512
    x = sequence.reshape(N, H)

    ke0 = knowledge_embeddings[:1]
    wvt = Wv.T.astype(jnp.bfloat16)
    wg2t = Wg[:, H:].T.astype(jnp.bfloat16)
    wg1t = Wg[:, :H].T.astype(jnp.bfloat16)
    wot = Wo.T.astype(jnp.bfloat16)
    bg2 = bg.reshape(1, H)
    bo2 = bo.reshape(1, H)
    gamma2 = ln_gamma.reshape(1, H)
    beta2 = ln_beta.reshape(1, H)

    full = lambda i: (0, 0)
    out = pl.pallas_call(
        _main_body,
        grid=(N // TM,),
        in_specs=[
            pl.BlockSpec((TM, H), lambda i: (i, 0)),
            pl.BlockSpec((H, H), full),
            pl.BlockSpec((H, H), full),
            pl.BlockSpec((H, H), full),
            pl.BlockSpec((H, H), full),
            pl.BlockSpec((1, H), full),
            pl.BlockSpec((1, H), full),
            pl.BlockSpec((1, H), full),
            pl.BlockSpec((1, H), full),
            pl.BlockSpec((1, H), full),
        ],
        out_specs=pl.BlockSpec((TM, H), lambda i: (i, 0)),
        out_shape=jax.ShapeDtypeStruct((N, H), jnp.float32),
        scratch_shapes=[
            pltpu.VMEM((1, H), jnp.float32),
            pltpu.VMEM((1, H), jnp.float32),
        ],
        interpret=interpret,
    )(x, wg1t, wot, wvt, wg2t, ke0, bg2, bo2, gamma2, beta2)

    return out.reshape(B, S, H)


# speedup vs baseline: 1.0381x; 1.0281x over previous
import functools

import jax
import jax.numpy as jnp
from jax.experimental import pallas as pl
from jax.experimental.pallas import tpu as pltpu


def _main_body(x_ref, wg1t_ref, wot_ref, wvt_ref, wg2t_ref, ke_ref, bg_ref,
               bo_ref, gamma_ref, beta_ref, out_ref, v_ref, c_ref):
    i = pl.program_id(0)

    @pl.when(i == 0)
    def _():
        v = jnp.dot(ke_ref[...].astype(jnp.bfloat16), wvt_ref[...],
                    preferred_element_type=jnp.float32)
        c = jnp.dot(v.astype(jnp.bfloat16), wg2t_ref[...],
                    preferred_element_type=jnp.float32) + bg_ref[...]
        v_ref[...] = v
        c_ref[...] = c

    tm = x_ref.shape[0]
    half = tm // 2

    def chain(x):
        xb = x.astype(jnp.bfloat16)
        g = jax.nn.sigmoid(
            jnp.dot(xb, wg1t_ref[...],
                    preferred_element_type=jnp.float32)
            + c_ref[...])
        vb = v_ref[...].astype(jnp.bfloat16)
        eb = xb + g.astype(jnp.bfloat16) * (vb - xb)
        p = jnp.dot(eb, wot_ref[...],
                    preferred_element_type=jnp.float32) + bo_ref[...]
        h = p.shape[-1]
        mu = jnp.sum(p, axis=-1, keepdims=True) * (1.0 / h)
        msq = jnp.sum(p * p, axis=-1, keepdims=True) * (1.0 / h)
        var = msq - mu * mu
        ln = ((p - mu) * jax.lax.rsqrt(var + 1e-5) * gamma_ref[...]
              + beta_ref[...])
        return ln + x

    out_ref[:half, :] = chain(x_ref[:half, :])
    out_ref[half:, :] = chain(x_ref[half:, :])


@functools.partial(jax.jit, static_argnames=("interpret",))
def kernel(sequence, knowledge_embeddings, Wq, Wk, Wv, Wg, bg, Wo, bo,
           ln_gamma, ln_beta, interpret=False):
    B, S, H = sequence.shape
    N = B * S
    TM = 512
    x = sequence.reshape(N, H)

    ke0 = knowledge_embeddings[:1]
    wvt = Wv.T.astype(jnp.bfloat16)
    wg2t = Wg[:, H:].T.astype(jnp.bfloat16)
    wg1t = Wg[:, :H].T.astype(jnp.bfloat16)
    wot = Wo.T.astype(jnp.bfloat16)
    bg2 = bg.reshape(1, H)
    bo2 = bo.reshape(1, H)
    gamma2 = ln_gamma.reshape(1, H)
    beta2 = ln_beta.reshape(1, H)

    full = lambda i: (0, 0)
    out = pl.pallas_call(
        _main_body,
        grid=(N // TM,),
        in_specs=[
            pl.BlockSpec((TM, H), lambda i: (i, 0)),
            pl.BlockSpec((H, H), full),
            pl.BlockSpec((H, H), full),
            pl.BlockSpec((H, H), full),
            pl.BlockSpec((H, H), full),
            pl.BlockSpec((1, H), full),
            pl.BlockSpec((1, H), full),
            pl.BlockSpec((1, H), full),
            pl.BlockSpec((1, H), full),
            pl.BlockSpec((1, H), full),
        ],
        out_specs=pl.BlockSpec((TM, H), lambda i: (i, 0)),
        out_shape=jax.ShapeDtypeStruct((N, H), jnp.float32),
        scratch_shapes=[
            pltpu.VMEM((1, H), jnp.float32),
            pltpu.VMEM((1, H), jnp.float32),
        ],
        interpret=interpret,
    )(x, wg1t, wot, wvt, wg2t, ke0, bg2, bo2, gamma2, beta2)

    return out.reshape(B, S, H)
